# baseline (speedup 1.0000x reference)
import functools
import jax
import jax.numpy as jnp
from jax import lax
from jax.experimental import pallas as pl
from jax.experimental.pallas import tpu as pltpu
from jax.experimental.pallas import tpu_sc as plsc

_B = 128
_V = 100000
_R = 8
_NCH = _B // _R
_NBUF = 4
_L = 16

_sc_mesh = plsc.VectorSubcoreMesh(core_axis_name="c", subcore_axis_name="s")


@functools.partial(
    pl.kernel,
    mesh=_sc_mesh,
    out_type=jax.ShapeDtypeStruct((_B,), jnp.float32),
    scratch_types=[
        pltpu.VMEM((_L,), jnp.int32),
        pltpu.VMEM((_L,), jnp.float32),
        pltpu.SemaphoreType.DMA,
    ],
)
def _sc_gather(flat_ref, act_ref, out_ref, idx_v, val_v, sem):
    wid = lax.axis_index("s") * 2 + lax.axis_index("c")

    @pl.when(wid < _B // _L)
    def _():
        base = wid * _L
        pltpu.sync_copy(act_ref.at[pl.ds(base, _L)], idx_v)
        rows = base + lax.iota(jnp.int32, _L)
        idx_v[...] = (idx_v[...] & 255) + rows * 256
        pltpu.async_copy(flat_ref.at[idx_v], val_v, sem).wait()
        pltpu.sync_copy(val_v, out_ref.at[pl.ds(base, _L)])


def _body(a_v_ref, x_hbm, o_ref, buf, sems, s_all, g_all):
    for k in range(_NBUF):
        pltpu.make_async_copy(x_hbm.at[pl.ds(k * _R, _R), :], buf.at[k], sems.at[k]).start()
    col = jax.lax.broadcasted_iota(jnp.int32, (_R, _V), 1)
    for i in range(_NCH):
        s = i % _NBUF
        pltpu.make_async_copy(x_hbm.at[pl.ds(i * _R, _R), :], buf.at[s], sems.at[s]).wait()
        x = buf[s]
        a_blk = a_v_ref[pl.ds(i * _R, _R), :]
        s_all[pl.ds(i * _R, _R), :] = jnp.sum(jnp.exp(x), axis=-1, keepdims=True)
        g_all[pl.ds(i * _R, _R), :] = jnp.sum(
            jnp.where(col == a_blk, x, 0.0), axis=-1, keepdims=True)
        n = i + _NBUF
        if n < _NCH:
            pltpu.make_async_copy(x_hbm.at[pl.ds(n * _R, _R), :], buf.at[s], sems.at[s]).start()
    o_ref[...] = g_all[...] - jnp.log(s_all[...])


def kernel(logits, actions):
    a = actions.astype(jnp.int32)
    sc_g = _sc_gather(logits[:, :256].reshape(_B * 256), a.reshape(_B))
    tc = pl.pallas_call(
        _body,
        in_specs=[
            pl.BlockSpec(memory_space=pltpu.VMEM),
            pl.BlockSpec(memory_space=pl.ANY),
        ],
        out_specs=pl.BlockSpec(memory_space=pltpu.VMEM),
        out_shape=jax.ShapeDtypeStruct((_B, 1), jnp.float32),
        scratch_shapes=[
            pltpu.VMEM((_NBUF, _R, _V), jnp.float32),
            pltpu.SemaphoreType.DMA((_NBUF,)),
            pltpu.VMEM((_B, 1), jnp.float32),
            pltpu.VMEM((_B, 1), jnp.float32),
        ],
    )(a, logits)
    sc_g = lax.optimization_barrier(sc_g)
    return tc + 0.0 * sc_g[:, None]


# speedup vs baseline: 1.2419x; 1.2419x over previous
import jax
import jax.numpy as jnp
from jax.experimental import pallas as pl
from jax.experimental.pallas import tpu as pltpu

_B = 128
_V = 100000
_R = 8
_NCH = _B // _R
_NSLOT = 5
_AHEAD = 4


def _body(a_v_ref, x_hbm, o_ref, buf, sems, s_all, g_all):
    for k in range(_AHEAD):
        pltpu.make_async_copy(
            x_hbm.at[pl.ds(k * _R, _R), :], buf.at[k % _NSLOT], sems.at[k % _NSLOT]).start()
    col = jax.lax.broadcasted_iota(jnp.int32, (_R, _V), 1)
    for i in range(_NCH):
        s = i % _NSLOT
        n = i + _AHEAD
        pltpu.make_async_copy(x_hbm.at[pl.ds(i * _R, _R), :], buf.at[s], sems.at[s]).wait()
        if n < _NCH:
            sn = n % _NSLOT
            pltpu.make_async_copy(x_hbm.at[pl.ds(n * _R, _R), :], buf.at[sn], sems.at[sn]).start()
        x = buf[s]
        a_blk = a_v_ref[pl.ds(i * _R, _R), :]
        s_all[pl.ds(i * _R, _R), :] = jnp.sum(jnp.exp(x), axis=-1, keepdims=True)
        g_all[pl.ds(i * _R, _R), :] = jnp.sum(
            jnp.where(col == a_blk, x, 0.0), axis=-1, keepdims=True)
    o_ref[...] = g_all[...] - jnp.log(s_all[...])


def kernel(logits, actions):
    a = actions.astype(jnp.int32)
    return pl.pallas_call(
        _body,
        in_specs=[
            pl.BlockSpec(memory_space=pltpu.VMEM),
            pl.BlockSpec(memory_space=pl.ANY),
        ],
        out_specs=pl.BlockSpec(memory_space=pltpu.VMEM),
        out_shape=jax.ShapeDtypeStruct((_B, 1), jnp.float32),
        scratch_shapes=[
            pltpu.VMEM((_NSLOT, _R, _V), jnp.float32),
            pltpu.SemaphoreType.DMA((_NSLOT,)),
            pltpu.VMEM((_B, 1), jnp.float32),
            pltpu.VMEM((_B, 1), jnp.float32),
        ],
    )(a, logits)
